# baseline (speedup 1.0000x reference)
import functools

import jax
import jax.numpy as jnp
from jax import lax
from jax.experimental import pallas as pl
from jax.experimental.pallas import tpu as pltpu
from jax.experimental.pallas import tpu_sc as plsc

N_CAT = 26
VOCAB = 100001
EMB = 32
N_NUM = 8
B = 1024
L = 50
ROWS = B * L
SLOTS = N_CAT + N_NUM
NC = 2
NS = 16
NW = NC * NS
RPW = ROWS // NW
CH = 32
NCH = RPW // CH
HALF = EMB // 2


def _sc_body(table, idx, xs, w, bv, out,
             idxbuf0, idxbuf1, rowbuf0, rowbuf1,
             xbuf, wbuf, bbuf,
             idx_sem0, idx_sem1, gat_sem0, gat_sem1, st_sem0, st_sem1):
    idxbufs = (idxbuf0, idxbuf1)
    rowbufs = (rowbuf0, rowbuf1)
    idx_sems = (idx_sem0, idx_sem1)
    gat_sems = (gat_sem0, gat_sem1)
    st_sems = (st_sem0, st_sem1)

    wid = lax.axis_index("s") * NC + lax.axis_index("c")
    row0 = wid * RPW

    for i in range(N_NUM):
        pltpu.sync_copy(xs.at[pl.ds(i * ROWS + row0, RPW)],
                        xbuf.at[pl.ds(i * RPW, RPW)])
    pltpu.sync_copy(w, wbuf)
    pltpu.sync_copy(bv, bbuf)

    def idx_copy(g, b):
        return pltpu.make_async_copy(
            idx.at[pl.ds(row0 + g * CH, CH), :], idxbufs[b], idx_sems[b])

    def slab_store(g, b):
        return pltpu.make_async_copy(
            rowbufs[b], out.at[pl.ds(row0 + g * CH, CH), :, :], st_sems[b])

    idx_copy(0, 0).start()
    idx_copy(1, 1).start()

    def chunk(g, b):
        @pl.when(g >= 2)
        def _():
            slab_store(g, b).wait()

        idx_copy(g, b).wait()

        def gat_copy(r):
            return pltpu.make_async_copy(
                table.at[idxbufs[b].at[r]],
                rowbufs[b].at[r, pl.ds(0, N_CAT), :],
                gat_sems[b])

        def fire(r, _):
            gat_copy(r).start()
            return 0

        lax.fori_loop(0, CH, fire, 0)

        nb = rowbufs[b]

        def num_group(h, _):
            r0 = h * HALF
            for i in range(N_NUM):
                wlo = wbuf[i, pl.ds(0, HALF)]
                whi = wbuf[i, pl.ds(HALF, HALF)]
                blo = bbuf[i, pl.ds(0, HALF)]
                bhi = bbuf[i, pl.ds(HALF, HALF)]
                xv = xbuf[pl.ds(i * RPW + g * CH + r0, HALF)]
                for r in range(HALF):
                    xs_ = jnp.broadcast_to(xv[r], (HALF,))
                    nb[r0 + r, N_CAT + i, pl.ds(0, HALF)] = xs_ * wlo + blo
                    nb[r0 + r, N_CAT + i, pl.ds(HALF, HALF)] = xs_ * whi + bhi
            return 0

        lax.fori_loop(0, CH // HALF, num_group, 0)

        def drain(r, _):
            gat_copy(r).wait()
            return 0

        lax.fori_loop(0, CH, drain, 0)
        slab_store(g, b).start()

        @pl.when(g + 2 < NCH)
        def _():
            idx_copy(g + 2, b).start()

    def outer(go, _):
        chunk(2 * go, 0)
        chunk(2 * go + 1, 1)
        return 0

    lax.fori_loop(0, NCH // 2, outer, 0)

    slab_store(NCH - 2, 0).wait()
    slab_store(NCH - 1, 1).wait()


_sc_call = functools.partial(
    pl.kernel,
    out_type=jax.ShapeDtypeStruct((ROWS, SLOTS, EMB), jnp.float32),
    mesh=plsc.VectorSubcoreMesh(core_axis_name="c", subcore_axis_name="s"),
    compiler_params=pltpu.CompilerParams(use_tc_tiling_on_sc=False),
    scratch_types=[
        pltpu.VMEM((CH, N_CAT), jnp.int32),
        pltpu.VMEM((CH, N_CAT), jnp.int32),
        pltpu.VMEM((CH, SLOTS, EMB), jnp.float32),
        pltpu.VMEM((CH, SLOTS, EMB), jnp.float32),
        pltpu.VMEM((N_NUM * RPW,), jnp.float32),
        pltpu.VMEM((N_NUM, EMB), jnp.float32),
        pltpu.VMEM((N_NUM, EMB), jnp.float32),
        pltpu.SemaphoreType.DMA,
        pltpu.SemaphoreType.DMA,
        pltpu.SemaphoreType.DMA,
        pltpu.SemaphoreType.DMA,
        pltpu.SemaphoreType.DMA,
        pltpu.SemaphoreType.DMA,
    ],
)(_sc_body)


def kernel(cat_features, num_features, mask, cat_tables, num_W, num_b):
    del mask
    offs = (jnp.arange(N_CAT, dtype=jnp.int32) * VOCAB).reshape(N_CAT, 1, 1)
    idx2d = jnp.transpose(cat_features.astype(jnp.int32) + offs,
                          (1, 2, 0)).reshape(ROWS, N_CAT)
    table2d = cat_tables.reshape(N_CAT * VOCAB, EMB)
    xflat = num_features.reshape(N_NUM * ROWS)
    out = _sc_call(table2d, idx2d, xflat, num_W, num_b)
    return out.reshape(B, L, SLOTS * EMB)


# speedup vs baseline: 2.3826x; 2.3826x over previous
import functools

import jax
import jax.numpy as jnp
from jax import lax
from jax.experimental import pallas as pl
from jax.experimental.pallas import tpu as pltpu
from jax.experimental.pallas import tpu_sc as plsc

N_CAT = 26
VOCAB = 100001
EMB = 32
N_NUM = 8
B = 1024
L = 50
ROWS = B * L
SLOTS = N_CAT + N_NUM
NC = 2
NS = 16
NW = NC * NS
RPW = ROWS // NW
CH = 32
NCH = RPW // CH
HALF = EMB // 2


def _sc_body(table, idx, xs, w, bv, out,
             idxbuf0, idxbuf1, fieldbuf0, fieldbuf1, numbuf0, numbuf1,
             xbuf, wbuf, bbuf,
             idx_sem0, idx_sem1, gat_sem0, gat_sem1, st_sem0, st_sem1):
    idxbufs = (idxbuf0, idxbuf1)
    fieldbufs = (fieldbuf0, fieldbuf1)
    numbufs = (numbuf0, numbuf1)
    idx_sems = (idx_sem0, idx_sem1)
    gat_sems = (gat_sem0, gat_sem1)
    st_sems = (st_sem0, st_sem1)

    wid = lax.axis_index("s") * NC + lax.axis_index("c")
    row0 = wid * RPW

    for i in range(N_NUM):
        pltpu.sync_copy(xs.at[pl.ds(i * ROWS + row0, RPW)],
                        xbuf.at[pl.ds(i * RPW, RPW)])
    pltpu.sync_copy(w, wbuf)
    pltpu.sync_copy(bv, bbuf)

    def idx_copy(g, b):
        return pltpu.make_async_copy(
            idx.at[:, pl.ds(row0 + g * CH, CH)], idxbufs[b], idx_sems[b])

    def cat_store(g, b, f):
        return pltpu.make_async_copy(
            fieldbufs[b].at[f], out.at[pl.ds(row0 + g * CH, CH), f, :],
            st_sems[b])

    def num_store(g, b):
        return pltpu.make_async_copy(
            numbufs[b], out.at[pl.ds(row0 + g * CH, CH), pl.ds(N_CAT, N_NUM), :],
            st_sems[b])

    def wait_stores(g, b):
        for f in range(N_CAT):
            cat_store(g, b, f).wait()
        num_store(g, b).wait()

    idx_copy(0, 0).start()
    idx_copy(1, 1).start()

    def chunk(g, b):
        @pl.when(g >= 2)
        def _():
            wait_stores(g, b)

        idx_copy(g, b).wait()

        def gat_copy(f):
            return pltpu.make_async_copy(
                table.at[f].at[idxbufs[b].at[f]],
                fieldbufs[b].at[f],
                gat_sems[b])

        for f in range(N_CAT):
            gat_copy(f).start()

        nb = numbufs[b]

        def num_group(h, _):
            r0 = h * HALF
            for i in range(N_NUM):
                wlo = wbuf[i, pl.ds(0, HALF)]
                whi = wbuf[i, pl.ds(HALF, HALF)]
                blo = bbuf[i, pl.ds(0, HALF)]
                bhi = bbuf[i, pl.ds(HALF, HALF)]
                xv = xbuf[pl.ds(i * RPW + g * CH + r0, HALF)]
                for r in range(HALF):
                    xs_ = jnp.broadcast_to(xv[r], (HALF,))
                    nb[r0 + r, i, pl.ds(0, HALF)] = xs_ * wlo + blo
                    nb[r0 + r, i, pl.ds(HALF, HALF)] = xs_ * whi + bhi
            return 0

        lax.fori_loop(0, CH // HALF, num_group, 0)

        for f in range(N_CAT):
            gat_copy(f).wait()

        for f in range(N_CAT):
            cat_store(g, b, f).start()
        num_store(g, b).start()

        @pl.when(g + 2 < NCH)
        def _():
            idx_copy(g + 2, b).start()

    def outer(go, _):
        chunk(2 * go, 0)
        chunk(2 * go + 1, 1)
        return 0

    lax.fori_loop(0, NCH // 2, outer, 0)

    wait_stores(NCH - 2, 0)
    wait_stores(NCH - 1, 1)


_sc_call = functools.partial(
    pl.kernel,
    out_type=jax.ShapeDtypeStruct((ROWS, SLOTS, EMB), jnp.float32),
    mesh=plsc.VectorSubcoreMesh(core_axis_name="c", subcore_axis_name="s"),
    compiler_params=pltpu.CompilerParams(use_tc_tiling_on_sc=False),
    scratch_types=[
        pltpu.VMEM((N_CAT, CH), jnp.int32),
        pltpu.VMEM((N_CAT, CH), jnp.int32),
        pltpu.VMEM((N_CAT, CH, EMB), jnp.float32),
        pltpu.VMEM((N_CAT, CH, EMB), jnp.float32),
        pltpu.VMEM((CH, N_NUM, EMB), jnp.float32),
        pltpu.VMEM((CH, N_NUM, EMB), jnp.float32),
        pltpu.VMEM((N_NUM * RPW,), jnp.float32),
        pltpu.VMEM((N_NUM, EMB), jnp.float32),
        pltpu.VMEM((N_NUM, EMB), jnp.float32),
        pltpu.SemaphoreType.DMA,
        pltpu.SemaphoreType.DMA,
        pltpu.SemaphoreType.DMA,
        pltpu.SemaphoreType.DMA,
        pltpu.SemaphoreType.DMA,
        pltpu.SemaphoreType.DMA,
    ],
)(_sc_body)


def kernel(cat_features, num_features, mask, cat_tables, num_W, num_b):
    del mask
    idx2 = cat_features.astype(jnp.int32).reshape(N_CAT, ROWS)
    xflat = num_features.reshape(N_NUM * ROWS)
    out = _sc_call(cat_tables, idx2, xflat, num_W, num_b)
    return out.reshape(B, L, SLOTS * EMB)


# speedup vs baseline: 5.1904x; 2.1785x over previous
import functools

import jax
import jax.numpy as jnp
from jax import lax
from jax.experimental import pallas as pl
from jax.experimental.pallas import tpu as pltpu
from jax.experimental.pallas import tpu_sc as plsc

N_CAT = 26
VOCAB = 100001
VOCABP = 100008
EMB = 32
N_NUM = 8
B = 1024
L = 50
ROWS = B * L
SLOTS = N_CAT + N_NUM
NC = 2
NS = 16
NW = NC * NS
RPW = ROWS // NW
CH = 32
NCH = RPW // CH
HALF = EMB // 2


def _sc_body(table, idx, xs, w, bv, out,
             idxbuf0, idxbuf1, fieldbuf0, fieldbuf1, numbuf0, numbuf1,
             xbuf, wbuf, bbuf,
             idx_sem0, idx_sem1, gat_sem0, gat_sem1, st_sem0, st_sem1):
    idxbufs = (idxbuf0, idxbuf1)
    fieldbufs = (fieldbuf0, fieldbuf1)
    numbufs = (numbuf0, numbuf1)
    idx_sems = (idx_sem0, idx_sem1)
    gat_sems = (gat_sem0, gat_sem1)
    st_sems = (st_sem0, st_sem1)

    wid = lax.axis_index("s") * NC + lax.axis_index("c")
    row0 = wid * RPW

    for i in range(N_NUM):
        pltpu.sync_copy(xs.at[pl.ds(i * ROWS + row0, RPW)],
                        xbuf.at[pl.ds(i * RPW, RPW)])
    pltpu.sync_copy(w, wbuf)
    pltpu.sync_copy(bv, bbuf)

    def idx_copy(g, b):
        return pltpu.make_async_copy(
            idx.at[:, pl.ds(row0 + g * CH, CH)], idxbufs[b], idx_sems[b])

    def cat_store(g, b, f):
        return pltpu.make_async_copy(
            fieldbufs[b].at[f], out.at[pl.ds(row0 + g * CH, CH), f, :],
            st_sems[b])

    def num_store(g, b):
        return pltpu.make_async_copy(
            numbufs[b], out.at[pl.ds(row0 + g * CH, CH), pl.ds(N_CAT, N_NUM), :],
            st_sems[b])

    def wait_stores(g, b):
        for f in range(N_CAT):
            cat_store(g, b, f).wait()
        num_store(g, b).wait()

    idx_copy(0, 0).start()
    idx_copy(1, 1).start()

    def chunk(g, b):
        @pl.when(g >= 2)
        def _():
            wait_stores(g, b)

        idx_copy(g, b).wait()

        def gat_copy(f):
            return pltpu.make_async_copy(
                table.at[f].at[idxbufs[b].at[f]],
                fieldbufs[b].at[f],
                gat_sems[b])

        for f in range(N_CAT):
            gat_copy(f).start()

        nb = numbufs[b]

        def num_group(h, _):
            r0 = h * HALF
            for i in range(N_NUM):
                wlo = wbuf[i, pl.ds(0, HALF)]
                whi = wbuf[i, pl.ds(HALF, HALF)]
                blo = bbuf[i, pl.ds(0, HALF)]
                bhi = bbuf[i, pl.ds(HALF, HALF)]
                xv = xbuf[pl.ds(i * RPW + g * CH + r0, HALF)]
                for r in range(HALF):
                    xs_ = jnp.broadcast_to(xv[r], (HALF,))
                    nb[r0 + r, i, pl.ds(0, HALF)] = xs_ * wlo + blo
                    nb[r0 + r, i, pl.ds(HALF, HALF)] = xs_ * whi + bhi
            return 0

        lax.fori_loop(0, CH // HALF, num_group, 0)

        for f in range(N_CAT):
            gat_copy(f).wait()

        for f in range(N_CAT):
            cat_store(g, b, f).start()
        num_store(g, b).start()

        @pl.when(g + 2 < NCH)
        def _():
            idx_copy(g + 2, b).start()

    def outer(go, _):
        chunk(2 * go, 0)
        chunk(2 * go + 1, 1)
        return 0

    lax.fori_loop(0, NCH // 2, outer, 0)

    wait_stores(NCH - 2, 0)
    wait_stores(NCH - 1, 1)


_sc_call = functools.partial(
    pl.kernel,
    out_type=jax.ShapeDtypeStruct((ROWS, SLOTS, EMB), jnp.float32),
    mesh=plsc.VectorSubcoreMesh(core_axis_name="c", subcore_axis_name="s"),
    compiler_params=pltpu.CompilerParams(use_tc_tiling_on_sc=False),
    scratch_types=[
        pltpu.VMEM((N_CAT, CH), jnp.int32),
        pltpu.VMEM((N_CAT, CH), jnp.int32),
        pltpu.VMEM((N_CAT, CH, EMB), jnp.float32),
        pltpu.VMEM((N_CAT, CH, EMB), jnp.float32),
        pltpu.VMEM((CH, N_NUM, EMB), jnp.float32),
        pltpu.VMEM((CH, N_NUM, EMB), jnp.float32),
        pltpu.VMEM((N_NUM * RPW,), jnp.float32),
        pltpu.VMEM((N_NUM, EMB), jnp.float32),
        pltpu.VMEM((N_NUM, EMB), jnp.float32),
        pltpu.SemaphoreType.DMA,
        pltpu.SemaphoreType.DMA,
        pltpu.SemaphoreType.DMA,
        pltpu.SemaphoreType.DMA,
        pltpu.SemaphoreType.DMA,
        pltpu.SemaphoreType.DMA,
    ],
)(_sc_body)


def kernel(cat_features, num_features, mask, cat_tables, num_W, num_b):
    del mask
    idx2 = cat_features.astype(jnp.int32).reshape(N_CAT, ROWS)
    xflat = num_features.reshape(N_NUM * ROWS)
    tabpad = jnp.pad(cat_tables, ((0, 0), (0, VOCABP - VOCAB), (0, 0)))
    out = _sc_call(tabpad, idx2, xflat, num_W, num_b)
    return out.reshape(B, L, SLOTS * EMB)
